# baseline (speedup 1.0000x reference)
import functools

import jax
import jax.numpy as jnp
from jax import lax
from jax.experimental import pallas as pl
from jax.experimental.pallas import tpu as pltpu
from jax.experimental.pallas import tpu_sc as plsc

N_ROWS = 32768
D = 2048
NC, NS = 2, 16
NW = NC * NS
B_PER_W = N_ROWS // NW
CHUNK = 16
N_CHUNKS = B_PER_W // CHUNK
NBUF = 3
SC_CHUNKS = 24
SC_ROWS = SC_CHUNKS * CHUNK
GRP = D // 16


def _shuffle(v, idx):
    return lax.gather(
        v,
        idx[:, None],
        lax.GatherDimensionNumbers(
            offset_dims=(), collapsed_slice_dims=(0,), start_index_map=(0,)
        ),
        (1,),
        mode=lax.GatherScatterMode.PROMISE_IN_BOUNDS,
    )


def _splat_reduce(v, op):
    lanes = lax.broadcasted_iota(jnp.int32, (16,), 0)
    for k in (8, 4, 2, 1):
        v = op(v, _shuffle(v, lanes ^ k))
    return v


def _sc_gather(table, idx_flat, tgt_flat):
    mesh = plsc.VectorSubcoreMesh(core_axis_name="c", subcore_axis_name="s")

    @functools.partial(
        pl.kernel,
        mesh=mesh,
        out_type=(
            jax.ShapeDtypeStruct((N_ROWS, D), jnp.float32),
            jax.ShapeDtypeStruct((N_ROWS,), jnp.float32),
            jax.ShapeDtypeStruct((N_ROWS,), jnp.float32),
            jax.ShapeDtypeStruct((N_ROWS,), jnp.float32),
        ),
        scratch_types=[
            pltpu.VMEM((B_PER_W,), jnp.int32),
            pltpu.VMEM((B_PER_W,), jnp.int32),
            pltpu.VMEM((B_PER_W,), jnp.float32),
            pltpu.VMEM((B_PER_W,), jnp.float32),
            pltpu.VMEM((B_PER_W,), jnp.float32),
        ]
        + [pltpu.VMEM((CHUNK, D), jnp.float32) for _ in range(NBUF)]
        + [pltpu.SemaphoreType.DMA for _ in range(2 * NBUF)],
    )
    def gather_kernel(
        table_hbm, idx_hbm, tgt_hbm,
        out_hbm, picked_hbm, m_hbm, s_hbm,
        idx_v, tgt_v, picked_v, m_v, s_v, *scratch
    ):
        bufs = scratch[:NBUF]
        sem_g = scratch[NBUF : 2 * NBUF]
        sem_s = scratch[2 * NBUF :]
        wid = lax.axis_index("s") * NC + lax.axis_index("c")
        base = wid * B_PER_W
        pltpu.sync_copy(idx_hbm.at[pl.ds(base, B_PER_W)], idx_v)
        pltpu.sync_copy(tgt_hbm.at[pl.ds(base, B_PER_W)], tgt_v)

        def gather_chunk(i, b):
            pltpu.async_copy(
                table_hbm.at[idx_v.at[pl.ds(i * CHUNK, CHUNK)]],
                bufs[b],
                sem_g[b],
            )

        def wait_gather(i, b):
            pltpu.make_async_copy(
                table_hbm.at[idx_v.at[pl.ds(i * CHUNK, CHUNK)]],
                bufs[b],
                sem_g[b],
            ).wait()

        def store_chunk(i, b):
            pltpu.async_copy(
                bufs[b],
                out_hbm.at[pl.ds(base + i * CHUNK, CHUNK)],
                sem_s[b],
            )

        def wait_store(i, b):
            pltpu.make_async_copy(
                bufs[b],
                out_hbm.at[pl.ds(base + i * CHUNK, CHUNK)],
                sem_s[b],
            ).wait()

        def pick_targets(j, b):
            tv = tgt_v[pl.ds(j * CHUNK, CHUNK)]
            lanes = lax.broadcasted_iota(jnp.int32, (16,), 0)
            pacc = jnp.zeros((16,), jnp.float32)
            for r in range(CHUNK):
                t_r = tv[r]
                v = bufs[b][r, pl.ds((t_r // 16) * 16, 16)]
                u = _shuffle(v, jnp.full((16,), t_r % 16, jnp.int32))
                pacc = jnp.where(lanes == r, u, pacc)
            picked_v[pl.ds(j * CHUNK, CHUNK)] = pacc

        def lse_chunk(j, b):
            lanes = lax.broadcasted_iota(jnp.int32, (16,), 0)

            def row_body(r, carry):
                mvec, svec = carry

                def mx(g, acc):
                    for u in range(8):
                        acc = jnp.maximum(
                            acc, bufs[b][r, pl.ds((g * 8 + u) * 16, 16)]
                        )
                    return acc

                macc = bufs[b][r, pl.ds(0, 16)]
                for u in range(1, 8):
                    macc = jnp.maximum(macc, bufs[b][r, pl.ds(u * 16, 16)])
                macc = lax.fori_loop(1, GRP // 8, mx, macc)
                m_spl = _splat_reduce(macc, jnp.maximum)

                def ex(g, acc):
                    for u in range(8):
                        acc = acc + jnp.exp(
                            bufs[b][r, pl.ds((g * 8 + u) * 16, 16)] - m_spl
                        )
                    return acc

                sacc = lax.fori_loop(
                    0, GRP // 8, ex, jnp.zeros((16,), jnp.float32)
                )
                s_spl = _splat_reduce(sacc, jnp.add)
                mvec = jnp.where(lanes == r, m_spl, mvec)
                svec = jnp.where(lanes == r, s_spl, svec)
                return mvec, svec

            mvec, svec = lax.fori_loop(
                0, CHUNK, row_body,
                (jnp.zeros((16,), jnp.float32), jnp.zeros((16,), jnp.float32)),
            )
            m_v[pl.ds(j * CHUNK, CHUNK)] = mvec
            s_v[pl.ds(j * CHUNK, CHUNK)] = svec

        gather_chunk(0, 0)

        N_MAIN = (N_CHUNKS - 1) // NBUF * NBUF

        def body(g, carry):
            for b in range(NBUF):
                j = g * NBUF + b
                bn = (b + 1) % NBUF
                wait_gather(j, b)
                store_chunk(j, b)
                pick_targets(j, b)

                @pl.when(j < SC_CHUNKS)
                def _():
                    lse_chunk(j, b)

                @pl.when(j - (NBUF - 1) >= 0)
                def _():
                    wait_store(j - (NBUF - 1), bn)

                gather_chunk(j + 1, bn)

            return carry

        lax.fori_loop(0, N_MAIN // NBUF, body, 0)

        for j in range(N_MAIN, N_CHUNKS):
            b = j % NBUF
            wait_gather(j, b)
            store_chunk(j, b)
            pick_targets(j, b)
        pltpu.sync_copy(picked_v, picked_hbm.at[pl.ds(base, B_PER_W)])
        pltpu.sync_copy(m_v, m_hbm.at[pl.ds(base, B_PER_W)])
        pltpu.sync_copy(s_v, s_hbm.at[pl.ds(base, B_PER_W)])
        for j in range(N_CHUNKS - NBUF, N_CHUNKS):
            wait_store(j, j % NBUF)

    return gather_kernel(table, idx_flat, tgt_flat)


TAIL_BLK = 128
TAIL_ROWS = B_PER_W - SC_ROWS
TAIL_BLKS = TAIL_ROWS // TAIL_BLK
BLK_PER_W = B_PER_W // TAIL_BLK
SC_BLKS = SC_ROWS // TAIL_BLK
N_TBLKS = N_ROWS // TAIL_BLK


def _tc_loss_kernel(pick_t_ref, pick_f_ref, m_ref, s_ref, logits_ref, acc_ref):
    w = pl.program_id(0)
    tb = pl.program_id(1)
    blk = logits_ref[...]
    m = jnp.max(blk, axis=1, keepdims=True)
    lse = jnp.log(jnp.sum(jnp.exp(blk - m), axis=1, keepdims=True)) + m
    part = jnp.sum(lse) - jnp.sum(pick_t_ref[0, 0, :])

    @pl.when(jnp.logical_and(w == 0, tb == 0))
    def _():
        acc_ref[0, 0] = 0.0

    acc_ref[0, 0] += part

    @pl.when(tb == 0)
    def _():
        ms = m_ref[0, 0, :]
        ss = s_ref[0, 0, :]
        ps = pick_f_ref[0, 0, :]
        cols = lax.broadcasted_iota(jnp.int32, (B_PER_W,), 0)
        nll = jnp.where(cols < SC_ROWS, ms + jnp.log(ss) - ps, 0.0)
        acc_ref[0, 0] += jnp.sum(nll)

    @pl.when(jnp.logical_and(w == NW - 1, tb == TAIL_BLKS - 1))
    def _():
        acc_ref[0, 0] = acc_ref[0, 0] / N_ROWS


def _tc_loss(logits, picked, m_arr, s_arr):
    pick_tail = picked.reshape(N_TBLKS, 1, TAIL_BLK)
    pick_full = picked.reshape(NW, 1, B_PER_W)
    m3 = m_arr.reshape(NW, 1, B_PER_W)
    s3 = s_arr.reshape(NW, 1, B_PER_W)
    acc = pl.pallas_call(
        _tc_loss_kernel,
        grid=(NW, TAIL_BLKS),
        in_specs=[
            pl.BlockSpec(
                (1, 1, TAIL_BLK),
                lambda w, tb: (w * BLK_PER_W + SC_BLKS + tb, 0, 0),
            ),
            pl.BlockSpec((1, 1, B_PER_W), lambda w, tb: (w, 0, 0)),
            pl.BlockSpec((1, 1, B_PER_W), lambda w, tb: (w, 0, 0)),
            pl.BlockSpec((1, 1, B_PER_W), lambda w, tb: (w, 0, 0)),
            pl.BlockSpec(
                (TAIL_BLK, D),
                lambda w, tb: (w * BLK_PER_W + SC_BLKS + tb, 0),
            ),
        ],
        out_specs=pl.BlockSpec(
            (1, 1), lambda w, tb: (0, 0), memory_space=pltpu.SMEM
        ),
        out_shape=jax.ShapeDtypeStruct((1, 1), jnp.float32),
    )(pick_tail, pick_full, m3, s3, logits)
    return acc[0, 0]


def kernel(x, targets, token_embedding_table):
    idx_flat = x.reshape(N_ROWS)
    tgt_flat = targets.reshape(N_ROWS)
    logits, picked, m_arr, s_arr = _sc_gather(
        token_embedding_table, idx_flat, tgt_flat
    )
    loss = _tc_loss(logits, picked, m_arr, s_arr)
    return (logits, loss)


# speedup vs baseline: 1.3474x; 1.3474x over previous
import functools

import jax
import jax.numpy as jnp
from jax import lax
from jax.experimental import pallas as pl
from jax.experimental.pallas import tpu as pltpu
from jax.experimental.pallas import tpu_sc as plsc

N_ROWS = 32768
D = 2048
NC, NS = 2, 16
NW = NC * NS
B_PER_W = N_ROWS // NW
CHUNK = 16
N_CHUNKS = B_PER_W // CHUNK
NBUF = 3


def _sc_gather(table, idx_flat, tgt_flat):
    mesh = plsc.VectorSubcoreMesh(core_axis_name="c", subcore_axis_name="s")

    @functools.partial(
        pl.kernel,
        mesh=mesh,
        out_type=(
            jax.ShapeDtypeStruct((N_ROWS, D), jnp.float32),
            jax.ShapeDtypeStruct((N_ROWS,), jnp.float32),
        ),
        scratch_types=[
            pltpu.VMEM((B_PER_W,), jnp.int32),
            pltpu.VMEM((B_PER_W,), jnp.int32),
            pltpu.VMEM((B_PER_W,), jnp.float32),
        ]
        + [pltpu.VMEM((CHUNK, D), jnp.float32) for _ in range(NBUF)]
        + [pltpu.SemaphoreType.DMA for _ in range(2 * NBUF)],
    )
    def gather_kernel(
        table_hbm, idx_hbm, tgt_hbm, out_hbm, picked_hbm,
        idx_v, tgt_v, picked_v, *scratch
    ):
        bufs = scratch[:NBUF]
        sem_g = scratch[NBUF : 2 * NBUF]
        sem_s = scratch[2 * NBUF :]
        wid = lax.axis_index("s") * NC + lax.axis_index("c")
        base = wid * B_PER_W
        pltpu.sync_copy(idx_hbm.at[pl.ds(base, B_PER_W)], idx_v)
        pltpu.sync_copy(tgt_hbm.at[pl.ds(base, B_PER_W)], tgt_v)

        def gather_chunk(i, b):
            pltpu.async_copy(
                table_hbm.at[idx_v.at[pl.ds(i * CHUNK, CHUNK)]],
                bufs[b],
                sem_g[b],
            )

        def wait_gather(i, b):
            pltpu.make_async_copy(
                table_hbm.at[idx_v.at[pl.ds(i * CHUNK, CHUNK)]],
                bufs[b],
                sem_g[b],
            ).wait()

        def store_chunk(i, b):
            pltpu.async_copy(
                bufs[b],
                out_hbm.at[pl.ds(base + i * CHUNK, CHUNK)],
                sem_s[b],
            )

        def wait_store(i, b):
            pltpu.make_async_copy(
                bufs[b],
                out_hbm.at[pl.ds(base + i * CHUNK, CHUNK)],
                sem_s[b],
            ).wait()

        def pick_targets(j, b):
            tv = tgt_v[pl.ds(j * CHUNK, CHUNK)]
            lanes = lax.broadcasted_iota(jnp.int32, (16,), 0)
            pacc = jnp.zeros((16,), jnp.float32)
            for r in range(CHUNK):
                t_r = tv[r]
                v = bufs[b][r, pl.ds((t_r // 16) * 16, 16)]
                u = lax.gather(
                    v,
                    jnp.full((16, 1), t_r % 16, jnp.int32),
                    lax.GatherDimensionNumbers(
                        offset_dims=(),
                        collapsed_slice_dims=(0,),
                        start_index_map=(0,),
                    ),
                    (1,),
                    mode=lax.GatherScatterMode.PROMISE_IN_BOUNDS,
                )
                pacc = jnp.where(lanes == r, u, pacc)
            picked_v[pl.ds(j * CHUNK, CHUNK)] = pacc

        gather_chunk(0, 0)

        N_MAIN = (N_CHUNKS - 1) // NBUF * NBUF

        def body(g, carry):
            for b in range(NBUF):
                j = g * NBUF + b
                bn = (b + 1) % NBUF
                wait_gather(j, b)
                store_chunk(j, b)
                pick_targets(j, b)

                @pl.when(j - (NBUF - 1) >= 0)
                def _():
                    wait_store(j - (NBUF - 1), bn)

                gather_chunk(j + 1, bn)

            return carry

        lax.fori_loop(0, N_MAIN // NBUF, body, 0)

        for j in range(N_MAIN, N_CHUNKS):
            b = j % NBUF
            wait_gather(j, b)
            store_chunk(j, b)
            pick_targets(j, b)
        pltpu.sync_copy(picked_v, picked_hbm.at[pl.ds(base, B_PER_W)])
        for j in range(N_CHUNKS - NBUF, N_CHUNKS):
            wait_store(j, j % NBUF)

    return gather_kernel(table, idx_flat, tgt_flat)


ROWS_BLK = 512
N_BLKS = N_ROWS // ROWS_BLK


def _tc_loss_kernel(picked_ref, logits_ref, acc_ref):
    i = pl.program_id(0)
    blk = logits_ref[...]
    m = jnp.max(blk, axis=1, keepdims=True)
    lse = jnp.log(jnp.sum(jnp.exp(blk - m), axis=1, keepdims=True)) + m
    part = jnp.sum(lse) - jnp.sum(picked_ref[0, 0, :])

    @pl.when(i == 0)
    def _():
        acc_ref[0, 0] = 0.0

    acc_ref[0, 0] += part

    @pl.when(i == N_BLKS - 1)
    def _():
        acc_ref[0, 0] = acc_ref[0, 0] / N_ROWS


def _tc_loss(logits, picked):
    tgt3 = picked.reshape(N_BLKS, 1, ROWS_BLK)
    acc = pl.pallas_call(
        _tc_loss_kernel,
        grid=(N_BLKS,),
        in_specs=[
            pl.BlockSpec((1, 1, ROWS_BLK), lambda i: (i, 0, 0)),
            pl.BlockSpec((ROWS_BLK, D), lambda i: (i, 0)),
        ],
        out_specs=pl.BlockSpec(
            (1, 1), lambda i: (0, 0), memory_space=pltpu.SMEM
        ),
        out_shape=jax.ShapeDtypeStruct((1, 1), jnp.float32),
    )(tgt3, logits)
    return acc[0, 0]


def kernel(x, targets, token_embedding_table):
    idx_flat = x.reshape(N_ROWS)
    tgt_flat = targets.reshape(N_ROWS)
    logits, picked = _sc_gather(token_embedding_table, idx_flat, tgt_flat)
    loss = _tc_loss(logits, picked)
    return (logits, loss)


# speedup vs baseline: 1.4373x; 1.0667x over previous
import functools

import jax
import jax.numpy as jnp
from jax import lax
from jax.experimental import pallas as pl
from jax.experimental.pallas import tpu as pltpu
from jax.experimental.pallas import tpu_sc as plsc

N_ROWS = 32768
D = 2048
NC, NS = 2, 16
NW = NC * NS
B_PER_W = N_ROWS // NW
CHUNK = 16
N_CHUNKS = B_PER_W // CHUNK
NBUF = 3


def _sc_gather(table, idx_flat, tgt_flat):
    mesh = plsc.VectorSubcoreMesh(core_axis_name="c", subcore_axis_name="s")

    @functools.partial(
        pl.kernel,
        mesh=mesh,
        out_type=(
            jax.ShapeDtypeStruct((N_ROWS, D), jnp.float32),
            jax.ShapeDtypeStruct((N_ROWS,), jnp.float32),
        ),
        scratch_types=[
            pltpu.VMEM((B_PER_W,), jnp.int32),
            pltpu.VMEM((B_PER_W,), jnp.int32),
            pltpu.VMEM((B_PER_W,), jnp.float32),
        ]
        + [pltpu.VMEM((CHUNK, D), jnp.float32) for _ in range(NBUF)]
        + [pltpu.SemaphoreType.DMA for _ in range(2 * NBUF)],
    )
    def gather_kernel(
        table_hbm, idx_hbm, tgt_hbm, out_hbm, picked_hbm,
        idx_v, tgt_v, picked_v, *scratch
    ):
        bufs = scratch[:NBUF]
        sem_g = scratch[NBUF : 2 * NBUF]
        sem_s = scratch[2 * NBUF :]
        wid = lax.axis_index("s") * NC + lax.axis_index("c")
        base = wid * B_PER_W
        pltpu.sync_copy(idx_hbm.at[pl.ds(base, B_PER_W)], idx_v)
        pltpu.sync_copy(tgt_hbm.at[pl.ds(base, B_PER_W)], tgt_v)

        def gather_chunk(i, b):
            pltpu.async_copy(
                table_hbm.at[idx_v.at[pl.ds(i * CHUNK, CHUNK)]],
                bufs[b],
                sem_g[b],
            )

        def wait_gather(i, b):
            pltpu.make_async_copy(
                table_hbm.at[idx_v.at[pl.ds(i * CHUNK, CHUNK)]],
                bufs[b],
                sem_g[b],
            ).wait()

        def store_chunk(i, b):
            pltpu.async_copy(
                bufs[b],
                out_hbm.at[pl.ds(base + i * CHUNK, CHUNK)],
                sem_s[b],
            )

        def wait_store(i, b):
            pltpu.make_async_copy(
                bufs[b],
                out_hbm.at[pl.ds(base + i * CHUNK, CHUNK)],
                sem_s[b],
            ).wait()

        def pick_targets(j, b):
            tv = tgt_v[pl.ds(j * CHUNK, CHUNK)]
            lanes = lax.broadcasted_iota(jnp.int32, (16,), 0)
            pacc = jnp.zeros((16,), jnp.float32)
            for r in range(CHUNK):
                t_r = tv[r]
                v = bufs[b][r, pl.ds((t_r // 16) * 16, 16)]
                u = lax.gather(
                    v,
                    jnp.full((16, 1), t_r % 16, jnp.int32),
                    lax.GatherDimensionNumbers(
                        offset_dims=(),
                        collapsed_slice_dims=(0,),
                        start_index_map=(0,),
                    ),
                    (1,),
                    mode=lax.GatherScatterMode.PROMISE_IN_BOUNDS,
                )
                pacc = jnp.where(lanes == r, u, pacc)
            picked_v[pl.ds(j * CHUNK, CHUNK)] = pacc

        gather_chunk(0, 0)

        N_MAIN = (N_CHUNKS - 1) // NBUF * NBUF

        def body(g, carry):
            for b in range(NBUF):
                j = g * NBUF + b
                bn = (b + 1) % NBUF
                wait_gather(j, b)
                store_chunk(j, b)
                pick_targets(j, b)

                @pl.when(j - (NBUF - 1) >= 0)
                def _():
                    wait_store(j - (NBUF - 1), bn)

                gather_chunk(j + 1, bn)

            return carry

        lax.fori_loop(0, N_MAIN // NBUF, body, 0)

        for j in range(N_MAIN, N_CHUNKS):
            b = j % NBUF
            wait_gather(j, b)
            store_chunk(j, b)
            pick_targets(j, b)
        pltpu.sync_copy(picked_v, picked_hbm.at[pl.ds(base, B_PER_W)])
        for j in range(N_CHUNKS - NBUF, N_CHUNKS):
            wait_store(j, j % NBUF)

    return gather_kernel(table, idx_flat, tgt_flat)


ROWS_BLK = 1024
N_BLKS = N_ROWS // ROWS_BLK


def _tc_loss_kernel(picked_ref, logits_ref, acc_ref):
    i = pl.program_id(0)
    blk = logits_ref[...]
    m = jnp.max(blk, axis=1, keepdims=True)
    lse = jnp.log(jnp.sum(jnp.exp(blk - m), axis=1, keepdims=True)) + m
    part = jnp.sum(lse) - jnp.sum(picked_ref[0, 0, :])

    @pl.when(i == 0)
    def _():
        acc_ref[0, 0] = 0.0

    acc_ref[0, 0] += part

    @pl.when(i == N_BLKS - 1)
    def _():
        acc_ref[0, 0] = acc_ref[0, 0] / N_ROWS


def _tc_loss(logits, picked):
    tgt3 = picked.reshape(N_BLKS, 1, ROWS_BLK)
    acc = pl.pallas_call(
        _tc_loss_kernel,
        grid=(N_BLKS,),
        in_specs=[
            pl.BlockSpec((1, 1, ROWS_BLK), lambda i: (i, 0, 0)),
            pl.BlockSpec((ROWS_BLK, D), lambda i: (i, 0)),
        ],
        out_specs=pl.BlockSpec(
            (1, 1), lambda i: (0, 0), memory_space=pltpu.SMEM
        ),
        out_shape=jax.ShapeDtypeStruct((1, 1), jnp.float32),
    )(tgt3, logits)
    return acc[0, 0]


def kernel(x, targets, token_embedding_table):
    idx_flat = x.reshape(N_ROWS)
    tgt_flat = targets.reshape(N_ROWS)
    logits, picked = _sc_gather(token_embedding_table, idx_flat, tgt_flat)
    loss = _tc_loss(logits, picked)
    return (logits, loss)


# speedup vs baseline: 1.4476x; 1.0071x over previous
import functools

import jax
import jax.numpy as jnp
from jax import lax
from jax.experimental import pallas as pl
from jax.experimental.pallas import tpu as pltpu
from jax.experimental.pallas import tpu_sc as plsc

N_ROWS = 32768
D = 2048
NC, NS = 2, 16
NW = NC * NS
B_PER_W = N_ROWS // NW
CHUNK = 16
N_CHUNKS = B_PER_W // CHUNK
NBUF = 3


def _sc_gather(table, idx_flat, tgt_flat):
    mesh = plsc.VectorSubcoreMesh(core_axis_name="c", subcore_axis_name="s")

    @functools.partial(
        pl.kernel,
        mesh=mesh,
        out_type=(
            jax.ShapeDtypeStruct((N_ROWS, D), jnp.float32),
            jax.ShapeDtypeStruct((N_ROWS,), jnp.float32),
        ),
        scratch_types=[
            pltpu.VMEM((B_PER_W,), jnp.int32),
            pltpu.VMEM((B_PER_W,), jnp.int32),
            pltpu.VMEM((B_PER_W,), jnp.float32),
        ]
        + [pltpu.VMEM((CHUNK, D), jnp.float32) for _ in range(NBUF)]
        + [pltpu.SemaphoreType.DMA for _ in range(2 * NBUF)],
    )
    def gather_kernel(
        table_hbm, idx_hbm, tgt_hbm, out_hbm, picked_hbm,
        idx_v, tgt_v, picked_v, *scratch
    ):
        bufs = scratch[:NBUF]
        sem_g = scratch[NBUF : 2 * NBUF]
        sem_s = scratch[2 * NBUF :]
        wid = lax.axis_index("s") * NC + lax.axis_index("c")
        base = wid * B_PER_W
        pltpu.sync_copy(idx_hbm.at[pl.ds(base, B_PER_W)], idx_v)
        pltpu.sync_copy(tgt_hbm.at[pl.ds(base, B_PER_W)], tgt_v)

        def gather_chunk(i, b):
            pltpu.async_copy(
                table_hbm.at[idx_v.at[pl.ds(i * CHUNK, CHUNK)]],
                bufs[b],
                sem_g[b],
            )

        def wait_gather(i, b):
            pltpu.make_async_copy(
                table_hbm.at[idx_v.at[pl.ds(i * CHUNK, CHUNK)]],
                bufs[b],
                sem_g[b],
            ).wait()

        def store_chunk(i, b):
            pltpu.async_copy(
                bufs[b],
                out_hbm.at[pl.ds(base + i * CHUNK, CHUNK)],
                sem_s[b],
            )

        def wait_store(i, b):
            pltpu.make_async_copy(
                bufs[b],
                out_hbm.at[pl.ds(base + i * CHUNK, CHUNK)],
                sem_s[b],
            ).wait()

        def pick_targets(j, b):
            tv = tgt_v[pl.ds(j * CHUNK, CHUNK)]
            lanes = lax.broadcasted_iota(jnp.int32, (16,), 0)
            pacc = jnp.zeros((16,), jnp.float32)
            for r in range(CHUNK):
                t_r = tv[r]
                v = bufs[b][r, pl.ds((t_r // 16) * 16, 16)]
                u = lax.gather(
                    v,
                    jnp.full((16, 1), t_r % 16, jnp.int32),
                    lax.GatherDimensionNumbers(
                        offset_dims=(),
                        collapsed_slice_dims=(0,),
                        start_index_map=(0,),
                    ),
                    (1,),
                    mode=lax.GatherScatterMode.PROMISE_IN_BOUNDS,
                )
                pacc = jnp.where(lanes == r, u, pacc)
            picked_v[pl.ds(j * CHUNK, CHUNK)] = pacc

        gather_chunk(0, 0)

        N_MAIN = (N_CHUNKS - 1) // NBUF * NBUF

        def body(g, carry):
            for b in range(NBUF):
                j = g * NBUF + b
                bn = (b + 1) % NBUF
                wait_gather(j, b)
                store_chunk(j, b)
                pick_targets(j, b)

                @pl.when(j - (NBUF - 1) >= 0)
                def _():
                    wait_store(j - (NBUF - 1), bn)

                gather_chunk(j + 1, bn)

            return carry

        lax.fori_loop(0, N_MAIN // NBUF, body, 0)

        for j in range(N_MAIN, N_CHUNKS):
            b = j % NBUF
            wait_gather(j, b)
            store_chunk(j, b)
            pick_targets(j, b)
        pltpu.sync_copy(picked_v, picked_hbm.at[pl.ds(base, B_PER_W)])
        for j in range(N_CHUNKS - NBUF, N_CHUNKS):
            wait_store(j, j % NBUF)

    return gather_kernel(table, idx_flat, tgt_flat)


ROWS_BLK = 2048
N_BLKS = N_ROWS // ROWS_BLK


def _tc_loss_kernel(picked_ref, logits_ref, acc_ref):
    i = pl.program_id(0)
    blk = logits_ref[...]
    m = jnp.max(blk, axis=1, keepdims=True)
    lse = jnp.log(jnp.sum(jnp.exp(blk - m), axis=1, keepdims=True)) + m
    part = jnp.sum(lse) - jnp.sum(picked_ref[0, 0, :])

    @pl.when(i == 0)
    def _():
        acc_ref[0, 0] = 0.0

    acc_ref[0, 0] += part

    @pl.when(i == N_BLKS - 1)
    def _():
        acc_ref[0, 0] = acc_ref[0, 0] / N_ROWS


def _tc_loss(logits, picked):
    tgt3 = picked.reshape(N_BLKS, 1, ROWS_BLK)
    acc = pl.pallas_call(
        _tc_loss_kernel,
        grid=(N_BLKS,),
        in_specs=[
            pl.BlockSpec((1, 1, ROWS_BLK), lambda i: (i, 0, 0)),
            pl.BlockSpec((ROWS_BLK, D), lambda i: (i, 0)),
        ],
        out_specs=pl.BlockSpec(
            (1, 1), lambda i: (0, 0), memory_space=pltpu.SMEM
        ),
        out_shape=jax.ShapeDtypeStruct((1, 1), jnp.float32),
    )(tgt3, logits)
    return acc[0, 0]


def kernel(x, targets, token_embedding_table):
    idx_flat = x.reshape(N_ROWS)
    tgt_flat = targets.reshape(N_ROWS)
    logits, picked = _sc_gather(token_embedding_table, idx_flat, tgt_flat)
    loss = _tc_loss(logits, picked)
    return (logits, loss)
